# baseline (speedup 1.0000x reference)
import functools

import jax
import jax.numpy as jnp
from jax import lax
from jax.experimental import pallas as pl
from jax.experimental.pallas import tpu as pltpu
from jax.experimental.pallas import tpu_sc as plsc

D = 768
E = 64
TOPK = 2
NG = 8
GS = E // NG
TG = 4
F = 256
FS = 512
SCALE = 2.5
T = 2048

TILE = 256
NT = E + (T * TOPK) // TILE
B = NT * TILE

NC = 2
NS = 16
NW = NC * NS
TPW = T // NW
VEC = 16


def _pack_halves(a):
    ai = lax.bitcast_convert_type(a[:, :D // 2], jnp.int32)
    bi = lax.bitcast_convert_type(a[:, D // 2:], jnp.int32)
    hi = (ai + 0x8000) & jnp.int32(-65536)
    lo = lax.shift_right_logical(bi + 0x8000, 16)
    return hi | lo


def _unpack_halves(w):
    fhi = lax.bitcast_convert_type(w & jnp.int32(-65536), jnp.float32)
    flo = lax.bitcast_convert_type(lax.shift_left(w, 16), jnp.float32)
    return jnp.concatenate([fhi, flo], axis=1)


def _router_body(x_ref, gw_ref, cb_ref, meta_ref, eot_ref, xp_ref):
    x = x_ref[...]
    xp_ref[...] = _pack_halves(x)
    logits = lax.dot_general(x, gw_ref[...], (((1,), (1,)), ((), ())),
                             preferred_element_type=jnp.float32)
    scores = jax.nn.sigmoid(logits)
    sfc = scores + cb_ref[...]

    lane = lax.broadcasted_iota(jnp.int32, (T, E), 1)
    grp_of_lane = lane // GS
    neg = jnp.float32(-jnp.inf)

    gs_full = jnp.zeros((T, E), jnp.float32)
    gcols = []
    m_incl = (lax.broadcasted_iota(jnp.int32, (E, E), 0) <=
              lax.broadcasted_iota(jnp.int32, (E, E), 1)).astype(jnp.float32)
    for g in range(NG):
        in_g = grp_of_lane == g
        seg = jnp.where(in_g, sfc, neg)
        m1 = jnp.max(seg, axis=1, keepdims=True)
        ismax = (seg == m1).astype(jnp.float32)
        csum = lax.dot_general(ismax, m_incl, (((1,), (0,)), ((), ())),
                               preferred_element_type=jnp.float32)
        first = (csum == 1.0) & (seg == m1)
        m2 = jnp.max(jnp.where(first, neg, seg), axis=1, keepdims=True)
        gsc = m1 + m2
        gcols.append(gsc)
        gs_full = jnp.where(in_g, gsc, gs_full)

    rank = jnp.zeros((T, E), jnp.int32)
    for gp in range(NG):
        sgp = gcols[gp]
        beats = (sgp > gs_full) | ((sgp == gs_full) & (gp < grp_of_lane))
        rank = rank + beats.astype(jnp.int32)
    em = rank < TG

    masked = jnp.where(em, sfc, neg)
    m1 = jnp.max(masked, axis=1, keepdims=True)
    id1 = jnp.min(jnp.where(masked == m1, lane, E), axis=1, keepdims=True)
    h1 = lane == id1
    masked2 = jnp.where(h1, neg, masked)
    m2 = jnp.max(masked2, axis=1, keepdims=True)
    id2 = jnp.min(jnp.where(masked2 == m2, lane, E), axis=1, keepdims=True)
    h2 = lane == id2

    w1 = jnp.sum(jnp.where(h1, scores, 0.0), axis=1, keepdims=True)
    w2 = jnp.sum(jnp.where(h2, scores, 0.0), axis=1, keepdims=True)
    s = w1 + w2 + 1e-20
    w1 = SCALE * w1 / s
    w2 = SCALE * w2 / s

    H = h1.astype(jnp.float32) + h2.astype(jnp.float32)
    BLK = 256
    tri = (lax.broadcasted_iota(jnp.int32, (BLK, BLK), 0) >
           lax.broadcasted_iota(jnp.int32, (BLK, BLK), 1)).astype(jnp.float32)
    run = jnp.zeros((8, E), jnp.float32)
    sparts = []
    for b in range(T // BLK):
        hb = lax.slice(H, (b * BLK, 0), ((b + 1) * BLK, E))
        sb = lax.dot_general(tri, hb, (((1,), (0,)), ((), ())),
                             preferred_element_type=jnp.float32)
        sparts.append(sb + run[0:1, :])
        run = run + jnp.sum(hb, axis=0, keepdims=True)
    S = jnp.concatenate(sparts, axis=0)
    counts = run

    tiles_per = jnp.floor((counts + (TILE - 1.0)) / TILE)
    m_strict = (lax.broadcasted_iota(jnp.int32, (E, E), 0) <
                lax.broadcasted_iota(jnp.int32, (E, E), 1)).astype(jnp.float32)
    tile_start = lax.dot_general(tiles_per, m_strict, (((1,), (0,)), ((), ())),
                                 preferred_element_type=jnp.float32)
    slot_start = tile_start[0:1, :] * TILE

    pos1 = jnp.sum(jnp.where(h1, slot_start + S, 0.0), axis=1, keepdims=True)
    pos2 = jnp.sum(jnp.where(h2, slot_start + S, 0.0), axis=1, keepdims=True)

    lane128 = lax.broadcasted_iota(jnp.int32, (T, 128), 1)
    meta = jnp.where(lane128 < 16, w1, 0.0)
    meta = jnp.where((lane128 >= 16) & (lane128 < 32), w2, meta)
    meta = jnp.where(lane128 == 32, pos1, meta)
    meta = jnp.where(lane128 == 33, pos2, meta)
    meta_ref[...] = meta

    jio = lax.broadcasted_iota(jnp.int32, (NT, E), 0).astype(jnp.float32)
    ts_b = jnp.broadcast_to(tile_start[0:1, :], (NT, E))
    eot = jnp.sum((ts_b <= jio).astype(jnp.int32), axis=1, keepdims=True) - 1
    ntiles = jnp.sum(tiles_per[0:1, :], axis=1, keepdims=True).astype(jnp.int32)
    eot2d = jnp.broadcast_to(eot, (NT, 128))
    lane_nt = lax.broadcasted_iota(jnp.int32, (NT, 128), 1)
    row_nt = lax.broadcasted_iota(jnp.int32, (NT, 128), 0)
    eot_ref[...] = jnp.where((lane_nt == 1) & (row_nt == 0),
                             jnp.broadcast_to(ntiles, (NT, 128)), eot2d)


def _router(x, gate_w, corr_bias_row):
    return pl.pallas_call(
        _router_body,
        out_shape=(
            jax.ShapeDtypeStruct((T, 128), jnp.float32),
            jax.ShapeDtypeStruct((NT, 128), jnp.int32),
            jax.ShapeDtypeStruct((T, D // 2), jnp.int32),
        ),
    )(x, gate_w, corr_bias_row)


@functools.lru_cache(maxsize=None)
def _build_sc_scatter():
    @functools.partial(
        pl.kernel,
        out_type=jax.ShapeDtypeStruct((B, D // 2), jnp.int32),
        mesh=plsc.VectorSubcoreMesh(core_axis_name="c", subcore_axis_name="s"),
        scratch_types=[
            pltpu.VMEM((TPW, D // 2), jnp.int32),
            pltpu.VMEM((TPW,), jnp.int32),
            pltpu.VMEM((TPW,), jnp.int32),
            pltpu.SemaphoreType.DMA,
        ],
    )
    def _sc_scatter(x_hbm, pos1_hbm, pos2_hbm, out_hbm, rows_v, idx1_v, idx2_v, sem):
        wid = lax.axis_index("s") * NC + lax.axis_index("c")
        base = wid * TPW
        pltpu.sync_copy(x_hbm.at[pl.ds(base, TPW)], rows_v)
        pltpu.sync_copy(pos1_hbm.at[pl.ds(base, TPW)], idx1_v)
        pltpu.sync_copy(pos2_hbm.at[pl.ds(base, TPW)], idx2_v)
        c1 = pltpu.async_copy(rows_v, out_hbm.at[idx1_v], sem)
        c2 = pltpu.async_copy(rows_v, out_hbm.at[idx2_v], sem)
        c1.wait()
        c2.wait()

    return _sc_scatter


def _gmm_body(eot_ref, nt_ref, xs_ref, wua_ref, wub_ref, wda_ref, wdb_ref,
              out_ref):
    @pl.when(pl.program_id(0) < nt_ref[0])
    def _():
        xb = _unpack_halves(xs_ref[...]).astype(jnp.bfloat16)
        out = None
        for wu_r, wd_r in ((wua_ref, wda_ref), (wub_ref, wdb_ref)):
            wu = wu_r[0].astype(jnp.bfloat16)
            h = lax.dot_general(xb, wu, (((1,), (0,)), ((), ())),
                                preferred_element_type=jnp.float32)
            h = jnp.square(jnp.maximum(h, 0.0)).astype(jnp.bfloat16)
            wd = wd_r[0].astype(jnp.bfloat16)
            o = lax.dot_general(h, wd, (((1,), (0,)), ((), ())),
                                preferred_element_type=jnp.float32)
            out = o if out is None else out + o
        out_ref[...] = _pack_halves(out)


def _gmm(eot, nt, x_sorted, w_up, w_down):
    def clamp(i, nt):
        return jnp.minimum(i, nt[0] - 1)

    grid_spec = pltpu.PrefetchScalarGridSpec(
        num_scalar_prefetch=2,
        grid=(NT,),
        in_specs=[
            pl.BlockSpec((TILE, D // 2), lambda i, eot, nt: (clamp(i, nt), 0)),
            pl.BlockSpec((1, D, F // 2),
                         lambda i, eot, nt: (eot[clamp(i, nt)], 0, 0)),
            pl.BlockSpec((1, D, F // 2),
                         lambda i, eot, nt: (eot[clamp(i, nt)], 0, 1)),
            pl.BlockSpec((1, F // 2, D),
                         lambda i, eot, nt: (eot[clamp(i, nt)], 0, 0)),
            pl.BlockSpec((1, F // 2, D),
                         lambda i, eot, nt: (eot[clamp(i, nt)], 1, 0)),
        ],
        out_specs=pl.BlockSpec((TILE, D // 2), lambda i, eot, nt: (clamp(i, nt), 0)),
    )
    return pl.pallas_call(
        _gmm_body,
        grid_spec=grid_spec,
        out_shape=jax.ShapeDtypeStruct((B, D // 2), jnp.int32),
    )(eot, nt, x_sorted, w_up, w_up, w_down, w_down)


def _combine_shared_body(x_ref, wu_ref, wd_ref, r1_ref, r2_ref, meta_ref,
                         out_ref):
    xb = _unpack_halves(x_ref[...]).astype(jnp.bfloat16)
    h = lax.dot_general(xb, wu_ref[...].astype(jnp.bfloat16),
                        (((1,), (0,)), ((), ())),
                        preferred_element_type=jnp.float32)
    h = jnp.square(jnp.maximum(h, 0.0)).astype(jnp.bfloat16)
    sh = lax.dot_general(h, wd_ref[...].astype(jnp.bfloat16),
                         (((1,), (0,)), ((), ())),
                         preferred_element_type=jnp.float32)
    meta = meta_ref[...]
    w1 = meta[:, 0:1]
    w2 = meta[:, VEC:VEC + 1]
    r1 = _unpack_halves(r1_ref[...])
    r2 = _unpack_halves(r2_ref[...])
    out_ref[...] = sh + w1 * r1 + w2 * r2


def _combine_shared(x, ws_up, ws_down, r1, r2, meta):
    SB = 256
    return pl.pallas_call(
        _combine_shared_body,
        grid=(T // SB,),
        in_specs=[
            pl.BlockSpec((SB, D // 2), lambda i: (i, 0)),
            pl.BlockSpec((D, FS), lambda i: (0, 0)),
            pl.BlockSpec((FS, D), lambda i: (0, 0)),
            pl.BlockSpec((SB, D // 2), lambda i: (i, 0)),
            pl.BlockSpec((SB, D // 2), lambda i: (i, 0)),
            pl.BlockSpec((SB, 128), lambda i: (i, 0)),
        ],
        out_specs=pl.BlockSpec((SB, D), lambda i: (i, 0)),
        out_shape=jax.ShapeDtypeStruct((T, D), jnp.float32),
    )(x, ws_up, ws_down, r1, r2, meta)


@functools.lru_cache(maxsize=None)
def _build_sc_gather():
    @functools.partial(
        pl.kernel,
        out_type=(jax.ShapeDtypeStruct((T, D // 2), jnp.int32),
                  jax.ShapeDtypeStruct((T, D // 2), jnp.int32)),
        mesh=plsc.VectorSubcoreMesh(core_axis_name="c", subcore_axis_name="s"),
        scratch_types=[
            pltpu.VMEM((TPW, D // 2), jnp.int32),
            pltpu.VMEM((TPW, D // 2), jnp.int32),
            pltpu.VMEM((TPW,), jnp.int32),
            pltpu.VMEM((TPW,), jnp.int32),
            pltpu.SemaphoreType.DMA,
        ],
    )
    def _sc_gather(rows_hbm, pos1_hbm, pos2_hbm, r1_out, r2_out,
                   r1_v, r2_v, idx1_v, idx2_v, sem):
        wid = lax.axis_index("s") * NC + lax.axis_index("c")
        base = wid * TPW
        pltpu.sync_copy(pos1_hbm.at[pl.ds(base, TPW)], idx1_v)
        pltpu.sync_copy(pos2_hbm.at[pl.ds(base, TPW)], idx2_v)
        c1 = pltpu.async_copy(rows_hbm.at[idx1_v], r1_v, sem)
        c2 = pltpu.async_copy(rows_hbm.at[idx2_v], r2_v, sem)
        c1.wait()
        c2.wait()
        pltpu.sync_copy(r1_v, r1_out.at[pl.ds(base, TPW)])
        pltpu.sync_copy(r2_v, r2_out.at[pl.ds(base, TPW)])

    return _sc_gather


def kernel(hidden_states, gate_w, corr_bias, w_up, w_down, ws_up, ws_down):
    x = hidden_states
    meta, eot2d, x_packed = _router(x, gate_w, corr_bias.reshape(1, E))
    pos1 = meta[:, 32].astype(jnp.int32)
    pos2 = meta[:, 33].astype(jnp.int32)
    eot = eot2d[:, 0]
    nt = eot2d[0, 1:2]

    x_sorted = _build_sc_scatter()(x_packed, pos1, pos2)
    rows = _gmm(eot, nt, x_sorted, w_up, w_down)
    r1, r2 = _build_sc_gather()(rows, pos1, pos2)
    return _combine_shared(x_packed, ws_up, ws_down, r1, r2, meta)


# speedup vs baseline: 1.0425x; 1.0425x over previous
import functools

import jax
import jax.numpy as jnp
from jax import lax
from jax.experimental import pallas as pl
from jax.experimental.pallas import tpu as pltpu
from jax.experimental.pallas import tpu_sc as plsc

D = 768
E = 64
TOPK = 2
NG = 8
GS = E // NG
TG = 4
F = 256
FS = 512
SCALE = 2.5
T = 2048

TILE = 256
NT = E + (T * TOPK) // TILE
B = NT * TILE

NC = 2
NS = 16
NW = NC * NS
TPW = T // NW
VEC = 16


def _pack_halves(a):
    ai = lax.bitcast_convert_type(a[:, :D // 2], jnp.int32)
    bi = lax.bitcast_convert_type(a[:, D // 2:], jnp.int32)
    hi = (ai + 0x8000) & jnp.int32(-65536)
    lo = lax.shift_right_logical(bi + 0x8000, 16)
    return hi | lo


def _unpack_halves(w):
    fhi = lax.bitcast_convert_type(w & jnp.int32(-65536), jnp.float32)
    flo = lax.bitcast_convert_type(lax.shift_left(w, 16), jnp.float32)
    return jnp.concatenate([fhi, flo], axis=1)


def _router_body(x_ref, gw_ref, cb_ref, meta_ref, eot_ref, xp_ref):
    x = x_ref[...]
    xp_ref[...] = _pack_halves(x)
    logits = lax.dot_general(x, gw_ref[...], (((1,), (1,)), ((), ())),
                             preferred_element_type=jnp.float32)
    scores = jax.nn.sigmoid(logits)
    sfc = scores + cb_ref[...]

    lane = lax.broadcasted_iota(jnp.int32, (T, E), 1)
    grp_of_lane = lane // GS
    neg = jnp.float32(-jnp.inf)

    gs_full = jnp.zeros((T, E), jnp.float32)
    gcols = []
    m_incl = (lax.broadcasted_iota(jnp.int32, (E, E), 0) <=
              lax.broadcasted_iota(jnp.int32, (E, E), 1)).astype(jnp.float32)
    for g in range(NG):
        in_g = grp_of_lane == g
        seg = jnp.where(in_g, sfc, neg)
        m1 = jnp.max(seg, axis=1, keepdims=True)
        ismax = (seg == m1).astype(jnp.float32)
        csum = lax.dot_general(ismax, m_incl, (((1,), (0,)), ((), ())),
                               preferred_element_type=jnp.float32)
        first = (csum == 1.0) & (seg == m1)
        m2 = jnp.max(jnp.where(first, neg, seg), axis=1, keepdims=True)
        gsc = m1 + m2
        gcols.append(gsc)
        gs_full = jnp.where(in_g, gsc, gs_full)

    rank = jnp.zeros((T, E), jnp.int32)
    for gp in range(NG):
        sgp = gcols[gp]
        beats = (sgp > gs_full) | ((sgp == gs_full) & (gp < grp_of_lane))
        rank = rank + beats.astype(jnp.int32)
    em = rank < TG

    masked = jnp.where(em, sfc, neg)
    m1 = jnp.max(masked, axis=1, keepdims=True)
    id1 = jnp.min(jnp.where(masked == m1, lane, E), axis=1, keepdims=True)
    h1 = lane == id1
    masked2 = jnp.where(h1, neg, masked)
    m2 = jnp.max(masked2, axis=1, keepdims=True)
    id2 = jnp.min(jnp.where(masked2 == m2, lane, E), axis=1, keepdims=True)
    h2 = lane == id2

    w1 = jnp.sum(jnp.where(h1, scores, 0.0), axis=1, keepdims=True)
    w2 = jnp.sum(jnp.where(h2, scores, 0.0), axis=1, keepdims=True)
    s = w1 + w2 + 1e-20
    w1 = SCALE * w1 / s
    w2 = SCALE * w2 / s

    H = h1.astype(jnp.float32) + h2.astype(jnp.float32)
    BLK = 256
    tri = (lax.broadcasted_iota(jnp.int32, (BLK, BLK), 0) >
           lax.broadcasted_iota(jnp.int32, (BLK, BLK), 1)).astype(jnp.float32)
    run = jnp.zeros((8, E), jnp.float32)
    sparts = []
    for b in range(T // BLK):
        hb = lax.slice(H, (b * BLK, 0), ((b + 1) * BLK, E))
        sb = lax.dot_general(tri, hb, (((1,), (0,)), ((), ())),
                             preferred_element_type=jnp.float32)
        sparts.append(sb + run[0:1, :])
        run = run + jnp.sum(hb, axis=0, keepdims=True)
    S = jnp.concatenate(sparts, axis=0)
    counts = run

    tiles_per = jnp.floor((counts + (TILE - 1.0)) / TILE)
    m_strict = (lax.broadcasted_iota(jnp.int32, (E, E), 0) <
                lax.broadcasted_iota(jnp.int32, (E, E), 1)).astype(jnp.float32)
    tile_start = lax.dot_general(tiles_per, m_strict, (((1,), (0,)), ((), ())),
                                 preferred_element_type=jnp.float32)
    slot_start = tile_start[0:1, :] * TILE

    pos1 = jnp.sum(jnp.where(h1, slot_start + S, 0.0), axis=1, keepdims=True)
    pos2 = jnp.sum(jnp.where(h2, slot_start + S, 0.0), axis=1, keepdims=True)

    lane128 = lax.broadcasted_iota(jnp.int32, (T, 128), 1)
    meta = jnp.where(lane128 < 16, w1, 0.0)
    meta = jnp.where((lane128 >= 16) & (lane128 < 32), w2, meta)
    meta = jnp.where(lane128 == 32, pos1, meta)
    meta = jnp.where(lane128 == 33, pos2, meta)
    meta_ref[...] = meta

    jio = lax.broadcasted_iota(jnp.int32, (NT, E), 0).astype(jnp.float32)
    ts_b = jnp.broadcast_to(tile_start[0:1, :], (NT, E))
    eot = jnp.sum((ts_b <= jio).astype(jnp.int32), axis=1, keepdims=True) - 1
    ntiles = jnp.sum(tiles_per[0:1, :], axis=1, keepdims=True).astype(jnp.int32)
    eot2d = jnp.broadcast_to(eot, (NT, 128))
    lane_nt = lax.broadcasted_iota(jnp.int32, (NT, 128), 1)
    row_nt = lax.broadcasted_iota(jnp.int32, (NT, 128), 0)
    eot_ref[...] = jnp.where((lane_nt == 1) & (row_nt == 0),
                             jnp.broadcast_to(ntiles, (NT, 128)), eot2d)


def _router(x, gate_w, corr_bias_row):
    return pl.pallas_call(
        _router_body,
        out_shape=(
            jax.ShapeDtypeStruct((T, 128), jnp.float32),
            jax.ShapeDtypeStruct((NT, 128), jnp.int32),
            jax.ShapeDtypeStruct((T, D // 2), jnp.int32),
        ),
    )(x, gate_w, corr_bias_row)


@functools.lru_cache(maxsize=None)
def _build_sc_scatter():
    @functools.partial(
        pl.kernel,
        out_type=jax.ShapeDtypeStruct((B, D // 2), jnp.int32),
        mesh=plsc.VectorSubcoreMesh(core_axis_name="c", subcore_axis_name="s"),
        scratch_types=[
            pltpu.VMEM((TPW, D // 2), jnp.int32),
            pltpu.VMEM((TPW,), jnp.int32),
            pltpu.VMEM((TPW,), jnp.int32),
            pltpu.SemaphoreType.DMA,
        ],
    )
    def _sc_scatter(x_hbm, pos1_hbm, pos2_hbm, out_hbm, rows_v, idx1_v, idx2_v, sem):
        wid = lax.axis_index("s") * NC + lax.axis_index("c")
        base = wid * TPW
        pltpu.sync_copy(x_hbm.at[pl.ds(base, TPW)], rows_v)
        pltpu.sync_copy(pos1_hbm.at[pl.ds(base, TPW)], idx1_v)
        pltpu.sync_copy(pos2_hbm.at[pl.ds(base, TPW)], idx2_v)
        c1 = pltpu.async_copy(rows_v, out_hbm.at[idx1_v], sem)
        c2 = pltpu.async_copy(rows_v, out_hbm.at[idx2_v], sem)
        c1.wait()
        c2.wait()

    return _sc_scatter


def _gmm_body(eot_ref, nt_ref, xs_ref, wu_ref, wd_ref, out_ref):
    @pl.when(pl.program_id(0) < nt_ref[0])
    def _():
        xb = _unpack_halves(xs_ref[...]).astype(jnp.bfloat16)
        wu = wu_ref[0].astype(jnp.bfloat16)
        h = lax.dot_general(xb, wu, (((1,), (0,)), ((), ())),
                            preferred_element_type=jnp.float32)
        h = jnp.square(jnp.maximum(h, 0.0)).astype(jnp.bfloat16)
        wd = wd_ref[0].astype(jnp.bfloat16)
        out = lax.dot_general(h, wd, (((1,), (0,)), ((), ())),
                              preferred_element_type=jnp.float32)
        out_ref[...] = _pack_halves(out)


def _gmm(eot, nt, x_sorted, w_up, w_down):
    def clamp(i, nt):
        return jnp.minimum(i, nt[0] - 1)

    grid_spec = pltpu.PrefetchScalarGridSpec(
        num_scalar_prefetch=2,
        grid=(NT,),
        in_specs=[
            pl.BlockSpec((TILE, D // 2), lambda i, eot, nt: (clamp(i, nt), 0)),
            pl.BlockSpec((1, D, F), lambda i, eot, nt: (eot[clamp(i, nt)], 0, 0)),
            pl.BlockSpec((1, F, D), lambda i, eot, nt: (eot[clamp(i, nt)], 0, 0)),
        ],
        out_specs=pl.BlockSpec((TILE, D // 2), lambda i, eot, nt: (clamp(i, nt), 0)),
    )
    return pl.pallas_call(
        _gmm_body,
        grid_spec=grid_spec,
        out_shape=jax.ShapeDtypeStruct((B, D // 2), jnp.int32),
    )(eot, nt, x_sorted, w_up, w_down)


def _combine_shared_body(x_ref, wu_ref, wd_ref, r1_ref, r2_ref, meta_ref,
                         out_ref):
    xb = _unpack_halves(x_ref[...]).astype(jnp.bfloat16)
    h = lax.dot_general(xb, wu_ref[...].astype(jnp.bfloat16),
                        (((1,), (0,)), ((), ())),
                        preferred_element_type=jnp.float32)
    h = jnp.square(jnp.maximum(h, 0.0)).astype(jnp.bfloat16)
    sh = lax.dot_general(h, wd_ref[...].astype(jnp.bfloat16),
                         (((1,), (0,)), ((), ())),
                         preferred_element_type=jnp.float32)
    meta = meta_ref[...]
    w1 = meta[:, 0:1]
    w2 = meta[:, VEC:VEC + 1]
    r1 = _unpack_halves(r1_ref[...])
    r2 = _unpack_halves(r2_ref[...])
    out_ref[...] = sh + w1 * r1 + w2 * r2


def _combine_shared(x, ws_up, ws_down, r1, r2, meta):
    SB = 256
    return pl.pallas_call(
        _combine_shared_body,
        grid=(T // SB,),
        in_specs=[
            pl.BlockSpec((SB, D // 2), lambda i: (i, 0)),
            pl.BlockSpec((D, FS), lambda i: (0, 0)),
            pl.BlockSpec((FS, D), lambda i: (0, 0)),
            pl.BlockSpec((SB, D // 2), lambda i: (i, 0)),
            pl.BlockSpec((SB, D // 2), lambda i: (i, 0)),
            pl.BlockSpec((SB, 128), lambda i: (i, 0)),
        ],
        out_specs=pl.BlockSpec((SB, D), lambda i: (i, 0)),
        out_shape=jax.ShapeDtypeStruct((T, D), jnp.float32),
    )(x, ws_up, ws_down, r1, r2, meta)


@functools.lru_cache(maxsize=None)
def _build_sc_gather():
    @functools.partial(
        pl.kernel,
        out_type=(jax.ShapeDtypeStruct((T, D // 2), jnp.int32),
                  jax.ShapeDtypeStruct((T, D // 2), jnp.int32)),
        mesh=plsc.VectorSubcoreMesh(core_axis_name="c", subcore_axis_name="s"),
        scratch_types=[
            pltpu.VMEM((TPW, D // 2), jnp.int32),
            pltpu.VMEM((TPW, D // 2), jnp.int32),
            pltpu.VMEM((TPW,), jnp.int32),
            pltpu.VMEM((TPW,), jnp.int32),
            pltpu.SemaphoreType.DMA,
        ],
    )
    def _sc_gather(rows_hbm, pos1_hbm, pos2_hbm, r1_out, r2_out,
                   r1_v, r2_v, idx1_v, idx2_v, sem):
        wid = lax.axis_index("s") * NC + lax.axis_index("c")
        base = wid * TPW
        pltpu.sync_copy(pos1_hbm.at[pl.ds(base, TPW)], idx1_v)
        pltpu.sync_copy(pos2_hbm.at[pl.ds(base, TPW)], idx2_v)
        c1 = pltpu.async_copy(rows_hbm.at[idx1_v], r1_v, sem)
        c2 = pltpu.async_copy(rows_hbm.at[idx2_v], r2_v, sem)
        c1.wait()
        c2.wait()
        pltpu.sync_copy(r1_v, r1_out.at[pl.ds(base, TPW)])
        pltpu.sync_copy(r2_v, r2_out.at[pl.ds(base, TPW)])

    return _sc_gather


def kernel(hidden_states, gate_w, corr_bias, w_up, w_down, ws_up, ws_down):
    x = hidden_states
    meta, eot2d, x_packed = _router(x, gate_w, corr_bias.reshape(1, E))
    pos1 = meta[:, 32].astype(jnp.int32)
    pos2 = meta[:, 33].astype(jnp.int32)
    eot = eot2d[:, 0]
    nt = eot2d[0, 1:2]

    x_sorted = _build_sc_scatter()(x_packed, pos1, pos2)
    rows = _gmm(eot, nt, x_sorted, w_up, w_down)
    r1, r2 = _build_sc_gather()(rows, pos1, pos2)
    return _combine_shared(x_packed, ws_up, ws_down, r1, r2, meta)
